# baseline (speedup 1.0000x reference)
import functools

import jax
import jax.numpy as jnp
from jax import lax
from jax.experimental import pallas as pl
from jax.experimental.pallas import tpu as pltpu
from jax.experimental.pallas import tpu_sc as plsc

_NC = 2
_NS = 16
_NW = _NC * _NS


def _sc_edge_scatter(table, ei, gather_row, n_rows_out, n_edges, batch):
    d = 2 * (table.shape[1] // 2)
    table = lax.bitcast_convert_type(
        table.reshape(table.shape[0], d // 2, 2), jnp.int32)
    scatter_row = 1 - gather_row
    e_per_w = n_edges // _NW
    n_iter = e_per_w // batch
    n_chunks = 10
    n_sub = n_iter // n_chunks
    rows_per_tile = -(-n_rows_out // (_NS * 8)) * 8
    n_pad = rows_per_tile * _NS

    mesh = plsc.VectorSubcoreMesh(core_axis_name="c", subcore_axis_name="s")

    nbuf = 2
    n_trip = n_sub // nbuf
    scratch = [
        pltpu.VMEM((n_sub, batch), jnp.int32),
        pltpu.VMEM((n_sub, batch), jnp.int32),
    ] + [pltpu.VMEM((batch, d // 2), jnp.int32) for _ in range(nbuf)] + [
        pltpu.VMEM((batch, d), jnp.float32) for _ in range(nbuf)] + [
        pltpu.VMEM_SHARED((n_pad, d), jnp.float32),
    ] + [pltpu.SemaphoreType.DMA for _ in range(2 * nbuf)]

    @functools.partial(
        pl.kernel,
        out_type=jax.ShapeDtypeStruct((_NC, n_pad, d), jnp.float32),
        mesh=mesh,
        scratch_types=scratch,
        compiler_params=pltpu.CompilerParams(use_tc_tiling_on_sc=False),
    )
    def k(table_hbm, ei_hbm, zeros_hbm, out_hbm,
          gi_v, si_v, bf0, bf1, fr0, fr1, acc_sh, sg0, sg1, ss0, ss1):
        bfs = (bf0, bf1)
        frs = (fr0, fr1)
        sgs = (sg0, sg1)
        sss = (ss0, ss1)
        c = lax.axis_index("c")
        s = lax.axis_index("s")
        wid = s * _NC + c
        mask = jnp.int32(-65536)

        def convert(bf, fr):
            for rr in range(batch):
                for ch in range(d // 32):
                    w = bf[rr, pl.ds(ch * 16, 16)]
                    fr[rr, pl.ds(ch * 32, 16)] = lax.bitcast_convert_type(
                        lax.shift_left(w, 16), jnp.float32)
                    fr[rr, pl.ds(ch * 32 + 16, 16)] = lax.bitcast_convert_type(
                        w & mask, jnp.float32)

        r0 = s * rows_per_tile
        pltpu.sync_copy(zeros_hbm.at[pl.ds(r0, rows_per_tile)],
                        acc_sh.at[pl.ds(r0, rows_per_tile)])
        plsc.subcore_barrier()

        def chunk(u, carry):
            pltpu.sync_copy(ei_hbm.at[gather_row, wid, u], gi_v)
            pltpu.sync_copy(ei_hbm.at[scatter_row, wid, u], si_v)
            for b in range(nbuf):
                pltpu.async_copy(table_hbm.at[gi_v.at[b]], bfs[b], sgs[b])

            def body(t, c2):
                j0 = t * nbuf
                for b in range(nbuf):
                    j = j0 + b
                    pltpu.make_async_copy(table_hbm.at[gi_v.at[0]], bfs[b], sgs[b]).wait()

                    @pl.when(j >= nbuf)
                    def _():
                        pltpu.make_async_copy(
                            frs[b], acc_sh.at[si_v.at[0]], sss[b]).wait()

                    convert(bfs[b], frs[b])
                    jn = j + nbuf

                    @pl.when(jn < n_sub)
                    def _():
                        pltpu.async_copy(table_hbm.at[gi_v.at[jn]], bfs[b], sgs[b])

                    pltpu.async_copy(frs[b], acc_sh.at[si_v.at[j]], sss[b], add=True)
                return c2

            lax.fori_loop(0, n_trip, body, 0)
            for b in range(nbuf):
                j = n_trip * nbuf + b
                if j < n_sub:
                    pltpu.make_async_copy(table_hbm.at[gi_v.at[0]], bfs[b], sgs[b]).wait()
                    pltpu.make_async_copy(
                        frs[b], acc_sh.at[si_v.at[0]], sss[b]).wait()
                    convert(bfs[b], frs[b])
                    pltpu.async_copy(frs[b], acc_sh.at[si_v.at[j]], sss[b], add=True)
            for b in range(nbuf):
                pltpu.make_async_copy(
                    frs[b], acc_sh.at[si_v.at[0]], sss[b]).wait()
            return carry

        lax.fori_loop(0, n_chunks, chunk, 0)
        plsc.subcore_barrier()

        pltpu.sync_copy(acc_sh.at[pl.ds(r0, rows_per_tile)],
                        out_hbm.at[c, pl.ds(r0, rows_per_tile)])

    return k(table, ei.reshape(2, _NW, n_chunks, n_sub, batch),
             jnp.zeros((n_pad, d), jnp.float32))


def _tc_linear(x, w, b, block_rows):
    r, d = x.shape

    def body(x_ref, w_ref, b_ref, o_ref):
        o_ref[...] = (
            jnp.dot(x_ref[...], w_ref[...].T, preferred_element_type=jnp.float32)
            + b_ref[...]
        ).astype(jnp.bfloat16)

    return pl.pallas_call(
        body,
        grid=(r // block_rows,),
        in_specs=[
            pl.BlockSpec((block_rows, d), lambda i: (i, 0)),
            pl.BlockSpec((d, d), lambda i: (0, 0)),
            pl.BlockSpec((1, d), lambda i: (0, 0)),
        ],
        out_specs=pl.BlockSpec((block_rows, d), lambda i: (i, 0)),
        out_shape=jax.ShapeDtypeStruct((r, d), jnp.bfloat16),
    )(x, w, b.reshape(1, d))


def _tc_fuse(acc, x_m, a1, a2, a3, bias, block_rows):
    m, d = x_m.shape
    d_out = a1.shape[1]

    def body(acc_ref, xm_ref, bd_ref, a1_ref, a2_ref, a3_ref, b_ref, o_ref):
        i = pl.program_id(0)
        hs = acc_ref[0] + acc_ref[1]
        xm = xm_ref[...]
        first = jnp.where(i == 0, 0.0, bd_ref[7:8, :])
        ag = jnp.concatenate([first, xm[:-1, :]], axis=0)
        o_ref[...] = (
            jnp.dot(hs, a1_ref[...], preferred_element_type=jnp.float32)
            + jnp.dot(xm, a2_ref[...], preferred_element_type=jnp.float32)
            + jnp.dot(ag, a3_ref[...], preferred_element_type=jnp.float32)
            + b_ref[...]
        ).astype(jnp.bfloat16)

    return pl.pallas_call(
        body,
        grid=(m // block_rows,),
        in_specs=[
            pl.BlockSpec((2, block_rows, d), lambda i: (0, i, 0)),
            pl.BlockSpec((block_rows, d), lambda i: (i, 0)),
            pl.BlockSpec((8, d), lambda i: (jnp.maximum(i * (block_rows // 8) - 1, 0), 0)),
            pl.BlockSpec((d, d_out), lambda i: (0, 0)),
            pl.BlockSpec((d, d_out), lambda i: (0, 0)),
            pl.BlockSpec((d, d_out), lambda i: (0, 0)),
            pl.BlockSpec((1, d_out), lambda i: (0, 0)),
        ],
        out_specs=pl.BlockSpec((block_rows, d_out), lambda i: (i, 0)),
        out_shape=jax.ShapeDtypeStruct((m, d_out), jnp.bfloat16),
    )(acc, x_m, x_m, a1, a2, a3, bias.reshape(1, d_out))


def _tc_sum_partials(p, r, block_rows):
    d = p.shape[2]

    def body(p_ref, o_ref):
        s = p_ref[0] + p_ref[1]
        br = s.shape[0]
        o_ref[...] = s.reshape(br, d // 32, 2, 16).transpose(0, 1, 3, 2).reshape(br, d)

    return pl.pallas_call(
        body,
        grid=(r // block_rows,),
        in_specs=[pl.BlockSpec((2, block_rows, d), lambda i: (0, i, 0))],
        out_specs=pl.BlockSpec((block_rows, d), lambda i: (i, 0)),
        out_shape=jax.ShapeDtypeStruct((r, d), jnp.float32),
    )(p)


def kernel(x_metrical, x, edge_index, batch, W_neigh, b_neigh, W_l, b_l, W_r,
           W_out, b_out, bn_weight, bn_bias):
    m, d = x_metrical.shape
    n = x.shape[0]
    e = edge_index.shape[1]

    ei = edge_index.astype(jnp.int32)

    h_neigh = _tc_linear(x, W_neigh, b_neigh, block_rows=1000)

    acc = _sc_edge_scatter(h_neigh, ei, 0, m, e, batch=100)

    inv = (1.0 + 1e-5) ** -0.5
    sc = (bn_weight * inv)[None, :]
    wo3t = W_out[:, 2 * d:].T
    perm = [32 * g + o for g in range(d // 32) for o in list(range(0, 32, 2)) + list(range(1, 32, 2))]
    a1 = (W_out[:, :d].T * sc)[jnp.array(perm), :]
    a2 = (W_out[:, d:2 * d].T + W_r.T @ wo3t) * sc
    a3 = (W_l.T @ wo3t) * sc
    bias = (b_l @ wo3t + b_out) * sc[0] + bn_bias
    h = _tc_fuse(acc, x_metrical, a1, a2, a3, bias, block_rows=1000)

    part = _sc_edge_scatter(h, ei, 1, n, e, batch=100)

    return _tc_sum_partials(part, n, block_rows=1000)


# speedup vs baseline: 2.3508x; 2.3508x over previous
import functools

import jax
import jax.numpy as jnp
from jax import lax
from jax.experimental import pallas as pl
from jax.experimental.pallas import tpu as pltpu
from jax.experimental.pallas import tpu_sc as plsc

_NC = 2
_NS = 16
_NW = _NC * _NS


def _sc_edge_scatter(table, ei, gather_row, n_rows_out, n_edges, batch):
    d = table.shape[1]
    scatter_row = 1 - gather_row
    e_per_w = n_edges // _NW
    n_iter = e_per_w // batch
    n_chunks = 5
    n_sub = n_iter // n_chunks
    rows_per_tile = -(-n_rows_out // (_NS * 8)) * 8
    n_pad = rows_per_tile * _NS

    mesh = plsc.VectorSubcoreMesh(core_axis_name="c", subcore_axis_name="s")

    nbuf = 4
    n_trip = n_sub // nbuf
    scratch = [
        pltpu.VMEM((n_sub, batch), jnp.int32),
        pltpu.VMEM((n_sub, batch), jnp.int32),
    ] + [pltpu.VMEM((batch, d), jnp.float32) for _ in range(nbuf)] + [
        pltpu.VMEM_SHARED((n_pad, d), jnp.float32),
    ] + [pltpu.SemaphoreType.DMA for _ in range(nbuf)]

    @functools.partial(
        pl.kernel,
        out_type=jax.ShapeDtypeStruct((_NC, n_pad, d), jnp.float32),
        mesh=mesh,
        scratch_types=scratch,
    )
    def k(table_hbm, ei_hbm, zeros_hbm, out_hbm,
          gi_v, si_v, rows0, rows1, rows2, rows3, acc_sh, sem0, sem1, sem2, sem3):
        rows = (rows0, rows1, rows2, rows3)
        sems = (sem0, sem1, sem2, sem3)
        c = lax.axis_index("c")
        s = lax.axis_index("s")
        wid = s * _NC + c

        r0 = s * rows_per_tile
        pltpu.sync_copy(zeros_hbm.at[pl.ds(r0, rows_per_tile)],
                        acc_sh.at[pl.ds(r0, rows_per_tile)])
        plsc.subcore_barrier()

        def chunk(u, carry):
            pltpu.sync_copy(ei_hbm.at[gather_row, wid, u], gi_v)
            pltpu.sync_copy(ei_hbm.at[scatter_row, wid, u], si_v)
            for b in range(nbuf):
                pltpu.async_copy(table_hbm.at[gi_v.at[b]], rows[b], sems[b])

            def body(t, c2):
                j0 = t * nbuf
                for b in range(nbuf):
                    j = j0 + b
                    pltpu.make_async_copy(table_hbm.at[gi_v.at[0]], rows[b], sems[b]).wait()
                    pltpu.sync_copy(rows[b], acc_sh.at[si_v.at[j]], add=True)
                    jn = j + nbuf

                    @pl.when(jn < n_sub)
                    def _():
                        pltpu.async_copy(table_hbm.at[gi_v.at[jn]], rows[b], sems[b])
                return c2

            lax.fori_loop(0, n_trip, body, 0)
            for b in range(nbuf):
                j = n_trip * nbuf + b
                if j < n_sub:
                    pltpu.make_async_copy(table_hbm.at[gi_v.at[0]], rows[b], sems[b]).wait()
                    pltpu.sync_copy(rows[b], acc_sh.at[si_v.at[j]], add=True)
            return carry

        lax.fori_loop(0, n_chunks, chunk, 0)
        plsc.subcore_barrier()

        pltpu.sync_copy(acc_sh.at[pl.ds(r0, rows_per_tile)],
                        out_hbm.at[c, pl.ds(r0, rows_per_tile)])

    return k(table, ei.reshape(2, _NW, n_chunks, n_sub, batch),
             jnp.zeros((n_pad, d), jnp.float32))


def _tc_linear(x, w, b, block_rows):
    r, d = x.shape

    def body(x_ref, w_ref, b_ref, o_ref):
        o_ref[...] = (
            jnp.dot(x_ref[...], w_ref[...].T, preferred_element_type=jnp.float32)
            + b_ref[...]
        )

    return pl.pallas_call(
        body,
        grid=(r // block_rows,),
        in_specs=[
            pl.BlockSpec((block_rows, d), lambda i: (i, 0)),
            pl.BlockSpec((d, d), lambda i: (0, 0)),
            pl.BlockSpec((1, d), lambda i: (0, 0)),
        ],
        out_specs=pl.BlockSpec((block_rows, d), lambda i: (i, 0)),
        out_shape=jax.ShapeDtypeStruct((r, d), jnp.float32),
    )(x, w, b.reshape(1, d))


def _tc_fuse(acc, x_m, a1, a2, a3, bias, block_rows):
    m, d = x_m.shape
    d_out = a1.shape[1]

    def body(acc_ref, xm_ref, bd_ref, a1_ref, a2_ref, a3_ref, b_ref, o_ref):
        i = pl.program_id(0)
        hs = acc_ref[0] + acc_ref[1]
        xm = xm_ref[...]
        first = jnp.where(i == 0, 0.0, bd_ref[7:8, :])
        ag = jnp.concatenate([first, xm[:-1, :]], axis=0)
        o_ref[...] = (
            jnp.dot(hs, a1_ref[...], preferred_element_type=jnp.float32)
            + jnp.dot(xm, a2_ref[...], preferred_element_type=jnp.float32)
            + jnp.dot(ag, a3_ref[...], preferred_element_type=jnp.float32)
            + b_ref[...]
        )

    return pl.pallas_call(
        body,
        grid=(m // block_rows,),
        in_specs=[
            pl.BlockSpec((2, block_rows, d), lambda i: (0, i, 0)),
            pl.BlockSpec((block_rows, d), lambda i: (i, 0)),
            pl.BlockSpec((8, d), lambda i: (jnp.maximum(i * (block_rows // 8) - 1, 0), 0)),
            pl.BlockSpec((d, d_out), lambda i: (0, 0)),
            pl.BlockSpec((d, d_out), lambda i: (0, 0)),
            pl.BlockSpec((d, d_out), lambda i: (0, 0)),
            pl.BlockSpec((1, d_out), lambda i: (0, 0)),
        ],
        out_specs=pl.BlockSpec((block_rows, d_out), lambda i: (i, 0)),
        out_shape=jax.ShapeDtypeStruct((m, d_out), jnp.float32),
    )(acc, x_m, x_m, a1, a2, a3, bias.reshape(1, d_out))


def _tc_sum_partials(p, r, block_rows):
    d = p.shape[2]

    def body(p_ref, o_ref):
        o_ref[...] = p_ref[0] + p_ref[1]

    return pl.pallas_call(
        body,
        grid=(r // block_rows,),
        in_specs=[pl.BlockSpec((2, block_rows, d), lambda i: (0, i, 0))],
        out_specs=pl.BlockSpec((block_rows, d), lambda i: (i, 0)),
        out_shape=jax.ShapeDtypeStruct((r, d), jnp.float32),
    )(p)


def kernel(x_metrical, x, edge_index, batch, W_neigh, b_neigh, W_l, b_l, W_r,
           W_out, b_out, bn_weight, bn_bias):
    m, d = x_metrical.shape
    n = x.shape[0]
    e = edge_index.shape[1]

    ei = edge_index.astype(jnp.int32)

    h_neigh = _tc_linear(x, W_neigh, b_neigh, block_rows=1000)

    acc = _sc_edge_scatter(h_neigh, ei, 0, m, e, batch=80)

    inv = (1.0 + 1e-5) ** -0.5
    sc = (bn_weight * inv)[None, :]
    wo3t = W_out[:, 2 * d:].T
    a1 = W_out[:, :d].T * sc
    a2 = (W_out[:, d:2 * d].T + W_r.T @ wo3t) * sc
    a3 = (W_l.T @ wo3t) * sc
    bias = (b_l @ wo3t + b_out) * sc[0] + bn_bias
    h = _tc_fuse(acc, x_metrical, a1, a2, a3, bias, block_rows=1000)

    part = _sc_edge_scatter(h, ei, 1, n, e, batch=80)

    return _tc_sum_partials(part, n, block_rows=1000)


# speedup vs baseline: 2.4170x; 1.0282x over previous
import functools

import jax
import jax.numpy as jnp
from jax import lax
from jax.experimental import pallas as pl
from jax.experimental.pallas import tpu as pltpu
from jax.experimental.pallas import tpu_sc as plsc

_NC = 2
_NS = 16
_NW = _NC * _NS


def _sc_edge_scatter(table, ei, gather_row, n_rows_out, n_edges, batch):
    d = table.shape[1]
    scatter_row = 1 - gather_row
    e_per_w = n_edges // _NW
    n_iter = e_per_w // batch
    n_chunks = 5
    n_sub = n_iter // n_chunks
    rows_per_tile = -(-n_rows_out // (_NS * 8)) * 8
    n_pad = rows_per_tile * _NS

    mesh = plsc.VectorSubcoreMesh(core_axis_name="c", subcore_axis_name="s")

    nbuf = 4
    n_trip = n_sub // nbuf
    scratch = [
        pltpu.VMEM((n_sub, batch), jnp.int32),
        pltpu.VMEM((n_sub, batch), jnp.int32),
    ] + [pltpu.VMEM((batch, d), jnp.float32) for _ in range(nbuf)] + [
        pltpu.VMEM_SHARED((n_pad, d), jnp.float32),
    ] + [pltpu.SemaphoreType.DMA for _ in range(nbuf)]

    @functools.partial(
        pl.kernel,
        out_type=jax.ShapeDtypeStruct((_NC, n_pad, d), jnp.float32),
        mesh=mesh,
        scratch_types=scratch,
    )
    def k(table_hbm, ei_hbm, zeros_hbm, out_hbm,
          gi_v, si_v, rows0, rows1, rows2, rows3, acc_sh, sem0, sem1, sem2, sem3):
        rows = (rows0, rows1, rows2, rows3)
        sems = (sem0, sem1, sem2, sem3)
        c = lax.axis_index("c")
        s = lax.axis_index("s")
        wid = s * _NC + c

        r0 = s * rows_per_tile
        pltpu.sync_copy(zeros_hbm.at[pl.ds(r0, rows_per_tile)],
                        acc_sh.at[pl.ds(r0, rows_per_tile)])
        plsc.subcore_barrier()

        def chunk(u, carry):
            pltpu.sync_copy(ei_hbm.at[gather_row, wid, u], gi_v)
            pltpu.sync_copy(ei_hbm.at[scatter_row, wid, u], si_v)
            for b in range(nbuf):
                pltpu.async_copy(table_hbm.at[gi_v.at[b]], rows[b], sems[b])

            def body(t, c2):
                j0 = t * nbuf
                for b in range(nbuf):
                    j = j0 + b
                    pltpu.make_async_copy(table_hbm.at[gi_v.at[0]], rows[b], sems[b]).wait()
                    pltpu.sync_copy(rows[b], acc_sh.at[si_v.at[j]], add=True)
                    jn = j + nbuf

                    @pl.when(jn < n_sub)
                    def _():
                        pltpu.async_copy(table_hbm.at[gi_v.at[jn]], rows[b], sems[b])
                return c2

            lax.fori_loop(0, n_trip, body, 0)
            for b in range(nbuf):
                j = n_trip * nbuf + b
                if j < n_sub:
                    pltpu.make_async_copy(table_hbm.at[gi_v.at[0]], rows[b], sems[b]).wait()
                    pltpu.sync_copy(rows[b], acc_sh.at[si_v.at[j]], add=True)
            return carry

        lax.fori_loop(0, n_chunks, chunk, 0)
        plsc.subcore_barrier()

        pltpu.sync_copy(acc_sh.at[pl.ds(r0, rows_per_tile)],
                        out_hbm.at[c, pl.ds(r0, rows_per_tile)])

    return k(table, ei.reshape(2, _NW, n_chunks, n_sub, batch),
             jnp.zeros((n_pad, d), jnp.float32))


def _tc_linear(x, w, b, block_rows):
    r, d = x.shape

    def body(x_ref, w_ref, b_ref, o_ref):
        o_ref[...] = (
            jnp.dot(x_ref[...], w_ref[...].T, preferred_element_type=jnp.float32)
            + b_ref[...]
        )

    return pl.pallas_call(
        body,
        grid=(r // block_rows,),
        in_specs=[
            pl.BlockSpec((block_rows, d), lambda i: (i, 0)),
            pl.BlockSpec((d, d), lambda i: (0, 0)),
            pl.BlockSpec((1, d), lambda i: (0, 0)),
        ],
        out_specs=pl.BlockSpec((block_rows, d), lambda i: (i, 0)),
        out_shape=jax.ShapeDtypeStruct((r, d), jnp.float32),
    )(x, w, b.reshape(1, d))


def _tc_fuse(acc, x_m, a1, a2, a3, bias, block_rows):
    m, d = x_m.shape
    d_out = a1.shape[1]

    def body(acc_ref, xm_ref, bd_ref, a1_ref, a2_ref, a3_ref, b_ref, o_ref):
        i = pl.program_id(0)
        hs = acc_ref[0] + acc_ref[1]
        xm = xm_ref[...]
        first = jnp.where(i == 0, 0.0, bd_ref[7:8, :])
        ag = jnp.concatenate([first, xm[:-1, :]], axis=0)
        o_ref[...] = (
            jnp.dot(hs, a1_ref[...], preferred_element_type=jnp.float32)
            + jnp.dot(xm, a2_ref[...], preferred_element_type=jnp.float32)
            + jnp.dot(ag, a3_ref[...], preferred_element_type=jnp.float32)
            + b_ref[...]
        )

    return pl.pallas_call(
        body,
        grid=(m // block_rows,),
        in_specs=[
            pl.BlockSpec((2, block_rows, d), lambda i: (0, i, 0)),
            pl.BlockSpec((block_rows, d), lambda i: (i, 0)),
            pl.BlockSpec((8, d), lambda i: (jnp.maximum(i * (block_rows // 8) - 1, 0), 0)),
            pl.BlockSpec((d, d_out), lambda i: (0, 0)),
            pl.BlockSpec((d, d_out), lambda i: (0, 0)),
            pl.BlockSpec((d, d_out), lambda i: (0, 0)),
            pl.BlockSpec((1, d_out), lambda i: (0, 0)),
        ],
        out_specs=pl.BlockSpec((block_rows, d_out), lambda i: (i, 0)),
        out_shape=jax.ShapeDtypeStruct((m, d_out), jnp.float32),
    )(acc, x_m, x_m, a1, a2, a3, bias.reshape(1, d_out))


def _tc_sum_partials(p, r, block_rows):
    d = p.shape[2]

    def body(p_ref, o_ref):
        o_ref[...] = p_ref[0] + p_ref[1]

    return pl.pallas_call(
        body,
        grid=(r // block_rows,),
        in_specs=[pl.BlockSpec((2, block_rows, d), lambda i: (0, i, 0))],
        out_specs=pl.BlockSpec((block_rows, d), lambda i: (i, 0)),
        out_shape=jax.ShapeDtypeStruct((r, d), jnp.float32),
    )(p)


def kernel(x_metrical, x, edge_index, batch, W_neigh, b_neigh, W_l, b_l, W_r,
           W_out, b_out, bn_weight, bn_bias):
    m, d = x_metrical.shape
    n = x.shape[0]
    e = edge_index.shape[1]

    ei = edge_index.astype(jnp.int32)

    h_neigh = _tc_linear(x, W_neigh, b_neigh, block_rows=2000)

    acc = _sc_edge_scatter(h_neigh, ei, 0, m, e, batch=80)

    inv = (1.0 + 1e-5) ** -0.5
    sc = (bn_weight * inv)[None, :]
    wo3t = W_out[:, 2 * d:].T
    a1 = W_out[:, :d].T * sc
    a2 = (W_out[:, d:2 * d].T + W_r.T @ wo3t) * sc
    a3 = (W_l.T @ wo3t) * sc
    bias = (b_l @ wo3t + b_out) * sc[0] + bn_bias
    h = _tc_fuse(acc, x_metrical, a1, a2, a3, bias, block_rows=2000)

    part = _sc_edge_scatter(h, ei, 1, n, e, batch=80)

    return _tc_sum_partials(part, n, block_rows=2000)


# speedup vs baseline: 2.4565x; 1.0163x over previous
import functools

import jax
import jax.numpy as jnp
from jax import lax
from jax.experimental import pallas as pl
from jax.experimental.pallas import tpu as pltpu
from jax.experimental.pallas import tpu_sc as plsc

_NC = 2
_NS = 16
_NW = _NC * _NS


def _sc_edge_scatter(table, ei, gather_row, n_rows_out, n_edges, batch):
    d = table.shape[1]
    scatter_row = 1 - gather_row
    e_per_w = n_edges // _NW
    n_iter = e_per_w // batch
    n_chunks = 5
    n_sub = n_iter // n_chunks
    rows_per_tile = -(-n_rows_out // (_NS * 8)) * 8
    n_pad = rows_per_tile * _NS

    mesh = plsc.VectorSubcoreMesh(core_axis_name="c", subcore_axis_name="s")

    nbuf = 4
    n_trip = n_sub // nbuf
    scratch = [
        pltpu.VMEM((n_sub, batch), jnp.int32),
        pltpu.VMEM((n_sub, batch), jnp.int32),
    ] + [pltpu.VMEM((batch, d), jnp.float32) for _ in range(nbuf)] + [
        pltpu.VMEM_SHARED((n_pad, d), jnp.float32),
    ] + [pltpu.SemaphoreType.DMA for _ in range(nbuf)]

    @functools.partial(
        pl.kernel,
        out_type=jax.ShapeDtypeStruct((_NC, n_pad, d), jnp.float32),
        mesh=mesh,
        scratch_types=scratch,
    )
    def k(table_hbm, ei_hbm, zeros_hbm, out_hbm,
          gi_v, si_v, rows0, rows1, rows2, rows3, acc_sh, sem0, sem1, sem2, sem3):
        rows = (rows0, rows1, rows2, rows3)
        sems = (sem0, sem1, sem2, sem3)
        c = lax.axis_index("c")
        s = lax.axis_index("s")
        wid = s * _NC + c

        r0 = s * rows_per_tile
        pltpu.sync_copy(zeros_hbm.at[pl.ds(r0, rows_per_tile)],
                        acc_sh.at[pl.ds(r0, rows_per_tile)])
        plsc.subcore_barrier()

        def chunk(u, carry):
            pltpu.sync_copy(ei_hbm.at[gather_row, wid, u], gi_v)
            pltpu.sync_copy(ei_hbm.at[scatter_row, wid, u], si_v)
            for b in range(nbuf):
                pltpu.async_copy(table_hbm.at[gi_v.at[b]], rows[b], sems[b])

            def body(t, c2):
                j0 = t * nbuf
                for b in range(nbuf):
                    j = j0 + b
                    pltpu.make_async_copy(table_hbm.at[gi_v.at[0]], rows[b], sems[b]).wait()
                    pltpu.sync_copy(rows[b], acc_sh.at[si_v.at[j]], add=True)
                    jn = j + nbuf

                    @pl.when(jn < n_sub)
                    def _():
                        pltpu.async_copy(table_hbm.at[gi_v.at[jn]], rows[b], sems[b])
                return c2

            lax.fori_loop(0, n_trip, body, 0)
            for b in range(nbuf):
                j = n_trip * nbuf + b
                if j < n_sub:
                    pltpu.make_async_copy(table_hbm.at[gi_v.at[0]], rows[b], sems[b]).wait()
                    pltpu.sync_copy(rows[b], acc_sh.at[si_v.at[j]], add=True)
            return carry

        lax.fori_loop(0, n_chunks, chunk, 0)
        plsc.subcore_barrier()

        pltpu.sync_copy(acc_sh.at[pl.ds(r0, rows_per_tile)],
                        out_hbm.at[c, pl.ds(r0, rows_per_tile)])

    return k(table, ei.reshape(2, _NW, n_chunks, n_sub, batch),
             jnp.zeros((n_pad, d), jnp.float32))


def _tc_linear(x, w, b, block_rows):
    r, d = x.shape

    def body(x_ref, w_ref, b_ref, o_ref):
        o_ref[...] = (
            jnp.dot(x_ref[...], w_ref[...].T, preferred_element_type=jnp.float32)
            + b_ref[...]
        )

    return pl.pallas_call(
        body,
        grid=(r // block_rows,),
        in_specs=[
            pl.BlockSpec((block_rows, d), lambda i: (i, 0)),
            pl.BlockSpec((d, d), lambda i: (0, 0)),
            pl.BlockSpec((1, d), lambda i: (0, 0)),
        ],
        out_specs=pl.BlockSpec((block_rows, d), lambda i: (i, 0)),
        out_shape=jax.ShapeDtypeStruct((r, d), jnp.float32),
    )(x, w, b.reshape(1, d))


def _tc_fuse(acc, x_m, a1, a2, a3, bias, block_rows):
    m, d = x_m.shape
    d_out = a1.shape[1]

    def body(acc_ref, xm_ref, bd_ref, a1_ref, a2_ref, a3_ref, b_ref, o_ref):
        i = pl.program_id(0)
        hs = acc_ref[0] + acc_ref[1]
        xm = xm_ref[...]
        first = jnp.where(i == 0, 0.0, bd_ref[7:8, :])
        ag = jnp.concatenate([first, xm[:-1, :]], axis=0)
        o_ref[...] = (
            jnp.dot(hs, a1_ref[...], preferred_element_type=jnp.float32)
            + jnp.dot(xm, a2_ref[...], preferred_element_type=jnp.float32)
            + jnp.dot(ag, a3_ref[...], preferred_element_type=jnp.float32)
            + b_ref[...]
        )

    return pl.pallas_call(
        body,
        grid=(m // block_rows,),
        in_specs=[
            pl.BlockSpec((2, block_rows, d), lambda i: (0, i, 0)),
            pl.BlockSpec((block_rows, d), lambda i: (i, 0)),
            pl.BlockSpec((8, d), lambda i: (jnp.maximum(i * (block_rows // 8) - 1, 0), 0)),
            pl.BlockSpec((d, d_out), lambda i: (0, 0)),
            pl.BlockSpec((d, d_out), lambda i: (0, 0)),
            pl.BlockSpec((d, d_out), lambda i: (0, 0)),
            pl.BlockSpec((1, d_out), lambda i: (0, 0)),
        ],
        out_specs=pl.BlockSpec((block_rows, d_out), lambda i: (i, 0)),
        out_shape=jax.ShapeDtypeStruct((m, d_out), jnp.float32),
    )(acc, x_m, x_m, a1, a2, a3, bias.reshape(1, d_out))


def _tc_sum_partials(p, r, block_rows):
    d = p.shape[2]

    def body(p_ref, o_ref):
        o_ref[...] = p_ref[0] + p_ref[1]

    return pl.pallas_call(
        body,
        grid=(r // block_rows,),
        in_specs=[pl.BlockSpec((2, block_rows, d), lambda i: (0, i, 0))],
        out_specs=pl.BlockSpec((block_rows, d), lambda i: (i, 0)),
        out_shape=jax.ShapeDtypeStruct((r, d), jnp.float32),
    )(p)


def kernel(x_metrical, x, edge_index, batch, W_neigh, b_neigh, W_l, b_l, W_r,
           W_out, b_out, bn_weight, bn_bias):
    m, d = x_metrical.shape
    n = x.shape[0]
    e = edge_index.shape[1]

    ei = edge_index.astype(jnp.int32)

    h_neigh = _tc_linear(x, W_neigh, b_neigh, block_rows=5000)

    acc = _sc_edge_scatter(h_neigh, ei, 0, m, e, batch=80)

    inv = (1.0 + 1e-5) ** -0.5
    sc = (bn_weight * inv)[None, :]
    wo3t = W_out[:, 2 * d:].T
    a1 = W_out[:, :d].T * sc
    a2 = (W_out[:, d:2 * d].T + W_r.T @ wo3t) * sc
    a3 = (W_l.T @ wo3t) * sc
    bias = (b_l @ wo3t + b_out) * sc[0] + bn_bias
    h = _tc_fuse(acc, x_metrical, a1, a2, a3, bias, block_rows=5000)

    part = _sc_edge_scatter(h, ei, 1, n, e, batch=80)

    return _tc_sum_partials(part, n, block_rows=5000)
